# baseline (speedup 1.0000x reference)
import functools

import jax
import jax.numpy as jnp
from jax import lax
from jax.experimental import pallas as pl
from jax.experimental.pallas import tpu as pltpu
from jax.experimental.pallas import tpu_sc as plsc

_CHUNK_ROWS = 64
_N_BUF = 2


def kernel(flat, row_splits):
    n_rows, n_feat = flat.shape
    mesh = plsc.VectorSubcoreMesh(core_axis_name="c", subcore_axis_name="s")
    n_workers = mesh.num_cores * mesh.num_subcores
    rows_per_w = n_rows // n_workers
    n_chunks = rows_per_w // _CHUNK_ROWS

    @functools.partial(
        pl.kernel,
        out_type=(
            jax.ShapeDtypeStruct(flat.shape, flat.dtype),
            jax.ShapeDtypeStruct(row_splits.shape, row_splits.dtype),
        ),
        mesh=mesh,
        scratch_types=[
            pltpu.VMEM((_N_BUF, _CHUNK_ROWS, n_feat), jnp.float32),
            pltpu.VMEM(row_splits.shape, row_splits.dtype),
            pltpu.SemaphoreType.DMA((_N_BUF,)),
            pltpu.SemaphoreType.DMA((_N_BUF,)),
            pltpu.SemaphoreType.DMA,
        ],
    )
    def sc_copy(flat_hbm, rs_hbm, flat_out, rs_out, bufs, rs_buf, in_sems,
                out_sems, rs_sem):
        cid = lax.axis_index("c")
        sid = lax.axis_index("s")
        wid = sid * mesh.num_cores + cid
        base = wid * rows_per_w

        @pl.when(wid == 0)
        def _():
            pltpu.make_async_copy(rs_hbm, rs_buf, rs_sem).start()
            pltpu.make_async_copy(rs_hbm, rs_buf, rs_sem).wait()
            pltpu.make_async_copy(rs_buf, rs_out, rs_sem).start()
            pltpu.make_async_copy(rs_buf, rs_out, rs_sem).wait()

        out_copies = [None] * _N_BUF
        for c in range(n_chunks):
            b = c % _N_BUF
            src = flat_hbm.at[pl.ds(base + c * _CHUNK_ROWS, _CHUNK_ROWS), :]
            dst = flat_out.at[pl.ds(base + c * _CHUNK_ROWS, _CHUNK_ROWS), :]
            if out_copies[b] is not None:
                out_copies[b].wait()
            c_in = pltpu.make_async_copy(src, bufs.at[b], in_sems.at[b])
            c_in.start()
            c_in.wait()
            c_out = pltpu.make_async_copy(bufs.at[b], dst, out_sems.at[b])
            c_out.start()
            out_copies[b] = c_out
        for b in range(_N_BUF):
            if out_copies[b] is not None:
                out_copies[b].wait()

    return sc_copy(flat, row_splits)


# speedup vs baseline: 1.0247x; 1.0247x over previous
import functools

import jax
import jax.numpy as jnp
from jax import lax
from jax.experimental import pallas as pl
from jax.experimental.pallas import tpu as pltpu
from jax.experimental.pallas import tpu_sc as plsc

_CHUNK_ROWS = 32
_N_BUF = 4


def kernel(flat, row_splits):
    n_rows, n_feat = flat.shape
    mesh = plsc.VectorSubcoreMesh(core_axis_name="c", subcore_axis_name="s")
    n_workers = mesh.num_cores * mesh.num_subcores
    rows_per_w = n_rows // n_workers
    n_chunks = rows_per_w // _CHUNK_ROWS

    @functools.partial(
        pl.kernel,
        out_type=(
            jax.ShapeDtypeStruct(flat.shape, flat.dtype),
            jax.ShapeDtypeStruct(row_splits.shape, row_splits.dtype),
        ),
        mesh=mesh,
        scratch_types=[
            pltpu.VMEM((_N_BUF, _CHUNK_ROWS, n_feat), jnp.float32),
            pltpu.VMEM(row_splits.shape, row_splits.dtype),
            pltpu.SemaphoreType.DMA((_N_BUF,)),
            pltpu.SemaphoreType.DMA((_N_BUF,)),
            pltpu.SemaphoreType.DMA,
        ],
    )
    def sc_copy(flat_hbm, rs_hbm, flat_out, rs_out, bufs, rs_buf, in_sems,
                out_sems, rs_sem):
        cid = lax.axis_index("c")
        sid = lax.axis_index("s")
        wid = sid * mesh.num_cores + cid
        base = wid * rows_per_w

        @pl.when(wid == 0)
        def _():
            pltpu.make_async_copy(rs_hbm, rs_buf, rs_sem).start()
            pltpu.make_async_copy(rs_hbm, rs_buf, rs_sem).wait()
            pltpu.make_async_copy(rs_buf, rs_out, rs_sem).start()
            pltpu.make_async_copy(rs_buf, rs_out, rs_sem).wait()

        def make_in(c):
            src = flat_hbm.at[pl.ds(base + c * _CHUNK_ROWS, _CHUNK_ROWS), :]
            return pltpu.make_async_copy(src, bufs.at[c % _N_BUF],
                                         in_sems.at[c % _N_BUF])

        def make_out(c):
            dst = flat_out.at[pl.ds(base + c * _CHUNK_ROWS, _CHUNK_ROWS), :]
            return pltpu.make_async_copy(bufs.at[c % _N_BUF], dst,
                                         out_sems.at[c % _N_BUF])

        in_copies = [None] * n_chunks
        out_copies = [None] * n_chunks
        in_copies[0] = make_in(0)
        in_copies[0].start()
        for c in range(n_chunks):
            nxt = c + 1
            if nxt < n_chunks:
                if nxt >= _N_BUF:
                    out_copies[nxt - _N_BUF].wait()
                in_copies[nxt] = make_in(nxt)
                in_copies[nxt].start()
            in_copies[c].wait()
            out_copies[c] = make_out(c)
            out_copies[c].start()
        for c in range(max(0, n_chunks - _N_BUF), n_chunks):
            out_copies[c].wait()

    return sc_copy(flat, row_splits)


# speedup vs baseline: 1.4946x; 1.4586x over previous
import jax
import jax.numpy as jnp
from jax.experimental import pallas as pl
from jax.experimental.pallas import tpu as pltpu

_CHUNK_ROWS = 1024
_N_BUF = 4


def _copy_kernel(flat_ref, rs_ref, flat_out, rs_out, bufs, in_sems, out_sems,
                 rs_sem):
    n_rows = flat_ref.shape[0]
    n_chunks = n_rows // _CHUNK_ROWS

    rs_in = pltpu.make_async_copy(rs_ref, rs_out, rs_sem)
    rs_in.start()

    def make_in(c):
        src = flat_ref.at[pl.ds(c * _CHUNK_ROWS, _CHUNK_ROWS), :]
        return pltpu.make_async_copy(src, bufs.at[c % _N_BUF],
                                     in_sems.at[c % _N_BUF])

    def make_out(c):
        dst = flat_out.at[pl.ds(c * _CHUNK_ROWS, _CHUNK_ROWS), :]
        return pltpu.make_async_copy(bufs.at[c % _N_BUF], dst,
                                     out_sems.at[c % _N_BUF])

    in_copies = [None] * n_chunks
    out_copies = [None] * n_chunks
    in_copies[0] = make_in(0)
    in_copies[0].start()
    for c in range(n_chunks):
        nxt = c + 1
        if nxt < n_chunks:
            if nxt >= _N_BUF:
                out_copies[nxt - _N_BUF].wait()
            in_copies[nxt] = make_in(nxt)
            in_copies[nxt].start()
        in_copies[c].wait()
        out_copies[c] = make_out(c)
        out_copies[c].start()
    for c in range(max(0, n_chunks - _N_BUF), n_chunks):
        out_copies[c].wait()
    rs_in.wait()


def kernel(flat, row_splits):
    n_rows, n_feat = flat.shape
    return pl.pallas_call(
        _copy_kernel,
        out_shape=(
            jax.ShapeDtypeStruct(flat.shape, flat.dtype),
            jax.ShapeDtypeStruct(row_splits.shape, row_splits.dtype),
        ),
        in_specs=[
            pl.BlockSpec(memory_space=pltpu.MemorySpace.HBM),
            pl.BlockSpec(memory_space=pltpu.MemorySpace.HBM),
        ],
        out_specs=(
            pl.BlockSpec(memory_space=pltpu.MemorySpace.HBM),
            pl.BlockSpec(memory_space=pltpu.MemorySpace.HBM),
        ),
        scratch_shapes=[
            pltpu.VMEM((_N_BUF, _CHUNK_ROWS, n_feat), jnp.float32),
            pltpu.SemaphoreType.DMA((_N_BUF,)),
            pltpu.SemaphoreType.DMA((_N_BUF,)),
            pltpu.SemaphoreType.DMA,
        ],
    )(flat, row_splits)
